# Initial kernel scaffold; baseline (speedup 1.0000x reference)
import jax
import jax.numpy as jnp
from jax.experimental import pallas as pl


def kernel(z_e, W):
    raise NotImplementedError("write your pallas kernel here")


# speedup vs baseline: 1.4804x; 1.4804x over previous
import jax
import jax.numpy as jnp
from jax.experimental import pallas as pl
from jax.experimental.pallas import tpu as pltpu

_BLK = 1024


def _zsq_tree(z):
    z2 = z * z
    c = ((z2[:, 0:8] + z2[:, 8:16]) + z2[:, 16:24]) + z2[:, 24:32]
    t1 = c[:, 0:4] + c[:, 4:8]
    t2 = t1[:, 0:2] + t1[:, 2:4]
    return t2[:, 0:1] + t2[:, 1:2]


def _vq_block(z_ref, w_ref, wt_ref, q_ref, loss_ref):
    z = z_ref[...]
    w = w_ref[...]
    wt = wt_ref[...]
    zsq = _zsq_tree(z)
    wsq = jnp.sum(w * w, axis=1)[None, :]
    s = jax.lax.dot_general(
        z, wt, dimension_numbers=(((1,), (0,)), ((), ())),
        preferred_element_type=jnp.float32)
    d2 = (zsq + wsq) - 2.0 * s
    dist = jnp.sqrt(jnp.maximum(d2, 0.0))
    m = jnp.min(dist, axis=1, keepdims=True)
    iota = jax.lax.broadcasted_iota(jnp.int32, dist.shape, 1)
    k = dist.shape[1]
    cand = jnp.where(dist == m, iota, k)
    idx = jnp.min(cand, axis=1)
    oh = jnp.where(iota == idx[:, None], 1.0, 0.0)
    q = jax.lax.dot_general(
        oh, w, dimension_numbers=(((1,), (0,)), ((), ())),
        preferred_element_type=jnp.float32)
    q_ref[...] = z + (q - z)
    part = jnp.sum((z - q) ** 2)[None, None]
    prev = jnp.where(pl.program_id(0) == 0, jnp.zeros((1, 1), jnp.float32),
                     loss_ref[...])
    loss_ref[...] = prev + part


def kernel(z_e, W):
    n, d = z_e.shape
    nk = W.shape[0]
    grid = n // _BLK
    Wt = W.T
    q, loss_sum = pl.pallas_call(
        _vq_block,
        grid=(grid,),
        in_specs=[
            pl.BlockSpec((_BLK, d), lambda i: (i, 0)),
            pl.BlockSpec((nk, d), lambda i: (0, 0)),
            pl.BlockSpec((d, nk), lambda i: (0, 0)),
        ],
        out_specs=[
            pl.BlockSpec((_BLK, d), lambda i: (i, 0)),
            pl.BlockSpec((1, 1), lambda i: (0, 0)),
        ],
        out_shape=[
            jax.ShapeDtypeStruct((n, d), jnp.float32),
            jax.ShapeDtypeStruct((1, 1), jnp.float32),
        ],
    )(z_e, W, Wt)
    vq_loss = (1.25 * loss_sum[0, 0]) / (n * d)
    return q, vq_loss
